# Initial kernel scaffold; baseline (speedup 1.0000x reference)
import jax
import jax.numpy as jnp
from jax.experimental import pallas as pl


def kernel(features_indices, weighted_features_indices, weighted_features_weights, ebc_tables, weighted_tables):
    raise NotImplementedError("write your pallas kernel here")


# speedup vs baseline: 1.5068x; 1.5068x over previous
import functools

import jax
import jax.numpy as jnp
from jax import lax
from jax.experimental import pallas as pl
from jax.experimental.pallas import tpu as pltpu
from jax.experimental.pallas import tpu_sc as plsc

_B, _F, _FW, _L, _V, _D = 4096, 2, 1, 20, 100000, 64
_NW = 32
_NC = 2
_BPW = _B // _NW
_CB = 4
_ROWS = _CB * _L
_NCH = _BPW // _CB
_NF = _F + _FW
_OD = _NF * _D


def _body(ebc_tab, w_tab, idx_hbm, wts_hbm, out_hbm, idx_v, wts_v, rows_v,
          out_v, sem):
    wid = lax.axis_index("s") * _NC + lax.axis_index("c")
    pltpu.sync_copy(idx_hbm.at[wid], idx_v)
    pltpu.sync_copy(wts_hbm.at[wid], wts_v)

    def feature(f, tab, weighted):
        def chunk_body(c, carry):
            pltpu.async_copy(tab.at[idx_v.at[f * _NCH + c]], rows_v,
                             sem).wait()
            if weighted:
                wv = [wts_v[c, pl.ds(g * 16, 16)] for g in range(_ROWS // 16)]
            for bag in range(_CB):
                accs = [jnp.zeros((16,), jnp.float32) for _ in range(4)]
                for l in range(_L):
                    r = bag * _L + l
                    if weighted:
                        w = wv[r // 16][r % 16]
                        for j in range(4):
                            accs[j] = accs[j] + rows_v[r, pl.ds(j * 16, 16)] * w
                    else:
                        for j in range(4):
                            accs[j] = accs[j] + rows_v[r, pl.ds(j * 16, 16)]
                ob = c * _CB + bag
                for j in range(4):
                    out_v[ob, pl.ds(f * _D + j * 16, 16)] = accs[j]
            return carry
        lax.fori_loop(0, _NCH, chunk_body, 0)

    feature(0, ebc_tab, False)
    feature(1, ebc_tab, False)
    feature(2, w_tab, True)
    pltpu.sync_copy(out_v, out_hbm.at[pl.ds(wid * _BPW, _BPW)])


@jax.jit
def _run(ebc_tab, w_tab, idx_all, wts):
    mesh = plsc.VectorSubcoreMesh(core_axis_name="c", subcore_axis_name="s")
    k = functools.partial(
        pl.kernel,
        mesh=mesh,
        out_type=jax.ShapeDtypeStruct((_B, _OD), jnp.float32),
        scratch_types=[
            pltpu.VMEM((_NF * _NCH, _ROWS), jnp.int32),
            pltpu.VMEM((_NCH, _ROWS), jnp.float32),
            pltpu.VMEM((_ROWS, _D), jnp.float32),
            pltpu.VMEM((_BPW, _OD), jnp.float32),
            pltpu.SemaphoreType.DMA,
        ],
        compiler_params=pltpu.CompilerParams(use_tc_tiling_on_sc=False),
    )(_body)
    return k(ebc_tab, w_tab, idx_all, wts)


def kernel(features_indices, weighted_features_indices,
           weighted_features_weights, ebc_tables, weighted_tables):
    ebc_tab = ebc_tables.reshape(_F * _V, _D)
    w_tab = weighted_tables.reshape(_FW * _V, _D)
    fi = features_indices.astype(jnp.int32).transpose(1, 0, 2)
    fi = fi + (jnp.arange(_F, dtype=jnp.int32) * _V)[:, None, None]
    fi = fi.reshape(_F, _NW, _BPW * _L).transpose(1, 0, 2)
    wi = weighted_features_indices.astype(jnp.int32).transpose(1, 0, 2)
    wi = wi.reshape(_FW, _NW, _BPW * _L).transpose(1, 0, 2)
    idx_all = jnp.concatenate([fi, wi], axis=1).reshape(_NW, _NF * _NCH, _ROWS)
    wts = weighted_features_weights.transpose(1, 0, 2).reshape(
        _NW, _NCH, _ROWS)
    return _run(ebc_tab, w_tab, idx_all, wts)


# speedup vs baseline: 1.6029x; 1.0638x over previous
import functools

import jax
import jax.numpy as jnp
from jax import lax
from jax.experimental import pallas as pl
from jax.experimental.pallas import tpu as pltpu
from jax.experimental.pallas import tpu_sc as plsc

_B, _F, _FW, _L, _V, _D = 4096, 2, 1, 20, 100000, 64
_NW = 32
_NC = 2
_BPW = _B // _NW
_ROWS = 80
_NE = _BPW * _F * _L // _ROWS
_NWC = _BPW * _FW * _L // _ROWS
_OD = (_F + _FW) * _D


def _body(ebc_tab, w_tab, fi_hbm, wi_hbm, wts_hbm, out_hbm,
          fi_v, wi_v, wts_v, rows_v, out_v, sem0, sem1):
    wid = lax.axis_index("s") * _NC + lax.axis_index("c")
    pltpu.sync_copy(fi_hbm.at[wid], fi_v)
    pltpu.sync_copy(wi_hbm.at[wid], wi_v)
    pltpu.sync_copy(wts_hbm.at[wid], wts_v)

    lane = lax.iota(jnp.int32, 16)
    offs = [jnp.where((lane + ph) % 40 >= 20, _V, 0).astype(jnp.int32)
            for ph in (0, 16, 32, 8, 24)]

    def off_body(r, carry):
        for i in range(5):
            sl = pl.ds(i * 16, 16)
            fi_v[r, sl] = fi_v[r, sl] + offs[i]
        return carry
    lax.fori_loop(0, _NE, off_body, 0)

    sems = (sem0, sem1)

    def start(tab, idx_row, buf):
        pltpu.async_copy(tab.at[idx_row], rows_v.at[buf], sems[buf])

    def drain(buf):
        pltpu.make_async_copy(ebc_tab.at[pl.ds(0, _ROWS)], rows_v.at[buf],
                              sems[buf]).wait()

    def accum_ebc(h, buf):
        for bag in range(2):
            for f in range(_F):
                accs = [jnp.zeros((16,), jnp.float32) for _ in range(4)]
                for l in range(_L):
                    r = bag * (_F * _L) + f * _L + l
                    for j in range(4):
                        accs[j] = accs[j] + rows_v[buf, r, pl.ds(j * 16, 16)]
                ob = 2 * h + bag
                for j in range(4):
                    out_v[ob, pl.ds(f * _D + j * 16, 16)] = accs[j]

    def accum_w(c, buf):
        wv = [wts_v[c, pl.ds(g * 16, 16)] for g in range(_ROWS // 16)]
        for bag in range(4):
            accs = [jnp.zeros((16,), jnp.float32) for _ in range(4)]
            for l in range(_L):
                r = bag * _L + l
                w = wv[r // 16][r % 16]
                for j in range(4):
                    accs[j] = accs[j] + rows_v[buf, r, pl.ds(j * 16, 16)] * w
            ob = 4 * c + bag
            for j in range(4):
                out_v[ob, pl.ds(_F * _D + j * 16, 16)] = accs[j]

    start(ebc_tab, fi_v.at[0], 0)

    def ebc_body(i, carry):
        a = 2 * i
        drain(0)
        start(ebc_tab, fi_v.at[a + 1], 1)
        accum_ebc(a, 0)
        drain(1)

        @pl.when(i < _NE // 2 - 1)
        def _():
            start(ebc_tab, fi_v.at[a + 2], 0)
        accum_ebc(a + 1, 1)
        return carry
    lax.fori_loop(0, _NE // 2, ebc_body, 0)

    start(w_tab, wi_v.at[0], 0)

    def w_body(i, carry):
        a = 2 * i
        drain(0)
        start(w_tab, wi_v.at[a + 1], 1)
        accum_w(a, 0)
        drain(1)

        @pl.when(i < _NWC // 2 - 1)
        def _():
            start(w_tab, wi_v.at[a + 2], 0)
        accum_w(a + 1, 1)
        return carry
    lax.fori_loop(0, _NWC // 2, w_body, 0)

    pltpu.sync_copy(out_v, out_hbm.at[pl.ds(wid * _BPW, _BPW)])


@jax.jit
def _run(ebc_tab, w_tab, fi_r, wi_r, wts_r):
    mesh = plsc.VectorSubcoreMesh(core_axis_name="c", subcore_axis_name="s")
    k = functools.partial(
        pl.kernel,
        mesh=mesh,
        out_type=jax.ShapeDtypeStruct((_B, _OD), jnp.float32),
        scratch_types=[
            pltpu.VMEM((_NE, _ROWS), jnp.int32),
            pltpu.VMEM((_NWC, _ROWS), jnp.int32),
            pltpu.VMEM((_NWC, _ROWS), jnp.float32),
            pltpu.VMEM((2, _ROWS, _D), jnp.float32),
            pltpu.VMEM((_BPW, _OD), jnp.float32),
            pltpu.SemaphoreType.DMA,
            pltpu.SemaphoreType.DMA,
        ],
        compiler_params=pltpu.CompilerParams(use_tc_tiling_on_sc=False),
    )(_body)
    return k(ebc_tab, w_tab, fi_r, wi_r, wts_r)


def kernel(features_indices, weighted_features_indices,
           weighted_features_weights, ebc_tables, weighted_tables):
    ebc_tab = ebc_tables.reshape(_F * _V, _D)
    w_tab = weighted_tables.reshape(_FW * _V, _D)
    fi_r = features_indices.astype(jnp.int32).reshape(_NW, _NE, _ROWS)
    wi_r = weighted_features_indices.astype(jnp.int32).reshape(
        _NW, _NWC, _ROWS)
    wts_r = weighted_features_weights.reshape(_NW, _NWC, _ROWS)
    return _run(ebc_tab, w_tab, fi_r, wi_r, wts_r)


# speedup vs baseline: 2.1523x; 1.3428x over previous
import functools

import jax
import jax.numpy as jnp
from jax import lax
from jax.experimental import pallas as pl
from jax.experimental.pallas import tpu as pltpu
from jax.experimental.pallas import tpu_sc as plsc

_B, _F, _FW, _L, _V, _D = 4096, 2, 1, 20, 100000, 64
_NF = _F + _FW
_NW = 32
_NP = _NF * _D
_PPW = _NP // _NW
_CB = 128
_NCH = _B // _CB
_NG = _CB // 16


def _body(ebc_t, w_t, idx_hbm, wts_hbm, out_hbm,
          row_v, idx_v0, idx_v1, wts_v0, wts_v1, out_row,
          sem_a, sem_b):
    idx_bufs = (idx_v0, idx_v1)
    wts_bufs = (wts_v0, wts_v1)
    slot = lax.axis_index("s")
    sc = lax.axis_index("c")
    wid = slot * 2 + sc

    sems = (sem_a, sem_b)

    def start_chunk(f, c, b, weighted):
        pltpu.async_copy(idx_hbm.at[f, c], idx_bufs[b], sems[b])
        if weighted:
            pltpu.async_copy(wts_hbm.at[c], wts_bufs[b], sems[b])

    def wait_chunk(f, c, b, weighted):
        pltpu.make_async_copy(idx_hbm.at[f, c], idx_bufs[b], sems[b]).wait()
        if weighted:
            pltpu.make_async_copy(wts_hbm.at[c], wts_bufs[b], sems[b]).wait()

    def accum_chunk(c, b, weighted):
        def group_body(g, carry):
            gl = pl.ds(pl.multiple_of(g * 16, 16), 16)
            acc = jnp.zeros((16,), jnp.float32)
            for l in range(_L):
                iv = idx_bufs[b][l, gl]
                val = plsc.load_gather(row_v, [iv])
                if weighted:
                    acc = acc + val * wts_bufs[b][l, gl]
                else:
                    acc = acc + val
            out_row[pl.ds(pl.multiple_of(c * _CB + g * 16, 16), 16)] = acc
            return carry
        lax.fori_loop(0, _NG, group_body, 0)

    def do_pair(k):
        p_lo = 32 * k
        f = p_lo // _D
        weighted = f == _F
        d = wid + 32 * (k % 2)
        tab = w_t if weighted else ebc_t
        pltpu.sync_copy(tab.at[0 if weighted else f, d], row_v)
        start_chunk(f, 0, 0, weighted)

        def step(i, carry):
            a = 2 * i
            wait_chunk(f, a, 0, weighted)
            start_chunk(f, a + 1, 1, weighted)
            accum_chunk(a, 0, weighted)
            wait_chunk(f, a + 1, 1, weighted)

            @pl.when(i < _NCH // 2 - 1)
            def _():
                start_chunk(f, a + 2, 0, weighted)
            accum_chunk(a + 1, 1, weighted)
            return carry
        lax.fori_loop(0, _NCH // 2, step, 0)
        pltpu.sync_copy(out_row, out_hbm.at[wid + 32 * k])

    for k in range(_PPW):
        do_pair(k)


@jax.jit
def _run(ebc_t, w_t, idx_all, wts):
    mesh = plsc.VectorSubcoreMesh(core_axis_name="c", subcore_axis_name="s")
    k = functools.partial(
        pl.kernel,
        mesh=mesh,
        out_type=jax.ShapeDtypeStruct((_NP, _B), jnp.float32),
        scratch_types=[
            pltpu.VMEM((_V,), jnp.float32),
            pltpu.VMEM((_L, _CB), jnp.int32),
            pltpu.VMEM((_L, _CB), jnp.int32),
            pltpu.VMEM((_L, _CB), jnp.float32),
            pltpu.VMEM((_L, _CB), jnp.float32),
            pltpu.VMEM((_B,), jnp.float32),
            pltpu.SemaphoreType.DMA,
            pltpu.SemaphoreType.DMA,
        ],
        compiler_params=pltpu.CompilerParams(needs_layout_passes=False),
    )(_body)
    return k(ebc_t, w_t, idx_all, wts)


def kernel(features_indices, weighted_features_indices,
           weighted_features_weights, ebc_tables, weighted_tables):
    ebc_t = ebc_tables.transpose(0, 2, 1)
    w_t = weighted_tables.transpose(0, 2, 1)
    it = features_indices.astype(jnp.int32).transpose(2, 1, 0)
    f0 = it[:, 0, :].reshape(_L, _NCH, _CB).transpose(1, 0, 2)
    f1 = it[:, 1, :].reshape(_L, _NCH, _CB).transpose(1, 0, 2)
    wi = weighted_features_indices.astype(jnp.int32).transpose(2, 1, 0)
    wi = wi[:, 0, :].reshape(_L, _NCH, _CB).transpose(1, 0, 2)
    idx_all = jnp.stack([f0, f1, wi])
    wts = weighted_features_weights.transpose(2, 1, 0)
    wts = wts[:, 0, :].reshape(_L, _NCH, _CB).transpose(1, 0, 2)
    out = _run(ebc_t, w_t, idx_all, wts)
    return out.T
